# baseline (speedup 1.0000x reference)
import jax
import jax.numpy as jnp
import numpy as np
from jax.experimental import pallas as pl

_D = 128
_DB = 64

_G1 = float(np.exp(-0.5))
_G2 = float(np.exp(-2.0))


def _box_hw(x, H, W):
    n = x.shape[0]
    zh = jnp.zeros((n, 1, W), jnp.float32)
    u = x + jnp.concatenate([zh, x[:, :-1, :]], axis=1) \
          + jnp.concatenate([x[:, 1:, :], zh], axis=1)
    zw = jnp.zeros((n, H, 1), jnp.float32)
    return u + jnp.concatenate([zw, u[:, :, :-1]], axis=2) \
             + jnp.concatenate([u[:, :, 1:], zw], axis=2)


def _splat_pool_kernel(depth_ref, xray_ref, out_ref):
    db = pl.program_id(1)
    nd = pl.num_programs(1)
    depth = depth_ref[0, 0]
    inten = xray_ref[0, 0]
    H, W = depth.shape
    di = jnp.clip((depth * (_D - 1)).astype(jnp.int32), 0, _D - 1)
    i27 = inten * (1.0 / 27.0)

    q = (db * _DB - 3) + jax.lax.broadcasted_iota(jnp.int32, (_DB + 6, 1, 1), 0)
    m = jnp.where(q == di[None, :, :], i27[None, :, :], 0.0)
    g = m[2:-2] + _G1 * (m[1:-3] + m[3:-1]) + _G2 * (m[:-4] + m[4:])
    t = g[1:-1] + g[:-2] + g[2:]
    out_ref[0] = _box_hw(t, H, W)

    @pl.when(db == 0)
    def _():
        out_ref[0, 0:1] -= _box_hw(g[0:1], H, W)

    @pl.when(db == nd - 1)
    def _():
        out_ref[0, _DB - 1:_DB] -= _box_hw(g[_DB + 1:_DB + 2], H, W)


def kernel(depth_map, x_ray):
    B, _, H, W = depth_map.shape
    out = pl.pallas_call(
        _splat_pool_kernel,
        grid=(B, _D // _DB),
        in_specs=[
            pl.BlockSpec((1, 1, H, W), lambda b, d: (b, 0, 0, 0)),
            pl.BlockSpec((1, 1, H, W), lambda b, d: (b, 0, 0, 0)),
        ],
        out_specs=pl.BlockSpec((1, _DB, H, W), lambda b, d: (b, d, 0, 0)),
        out_shape=jax.ShapeDtypeStruct((B, _D, H, W), jnp.float32),
    )(depth_map, x_ray)
    return out[:, None]


# speedup vs baseline: 1.0341x; 1.0341x over previous
import jax
import jax.numpy as jnp
import numpy as np
from jax.experimental import pallas as pl

_D = 128

_G1 = float(np.exp(-0.5))
_G2 = float(np.exp(-2.0))


def _splat_pool_kernel(depth_ref, xray_ref, out_ref):
    depth = depth_ref[0, 0]
    inten = xray_ref[0, 0]
    H, W = depth.shape
    di = jnp.clip((depth * (_D - 1)).astype(jnp.int32), 0, _D - 1)
    i27 = inten * (1.0 / 27.0)

    q = jax.lax.broadcasted_iota(jnp.int32, (_D, 1, 1), 0)
    m = jnp.where(q == di[None, :, :], i27[None, :, :], 0.0)

    z1 = jnp.zeros((1, H, W), jnp.float32)
    z2 = jnp.zeros((2, H, W), jnp.float32)
    mm1 = jnp.concatenate([z1, m[:-1]], axis=0)
    mp1 = jnp.concatenate([m[1:], z1], axis=0)
    mm2 = jnp.concatenate([z2, m[:-2]], axis=0)
    mp2 = jnp.concatenate([m[2:], z2], axis=0)
    g = m + _G1 * (mm1 + mp1) + _G2 * (mm2 + mp2)

    t = g + jnp.concatenate([z1, g[:-1]], axis=0) \
          + jnp.concatenate([g[1:], z1], axis=0)

    zh = jnp.zeros((_D, 1, W), jnp.float32)
    u = t + jnp.concatenate([zh, t[:, :-1, :]], axis=1) \
          + jnp.concatenate([t[:, 1:, :], zh], axis=1)

    zw = jnp.zeros((_D, H, 1), jnp.float32)
    out_ref[0] = u + jnp.concatenate([zw, u[:, :, :-1]], axis=2) \
                   + jnp.concatenate([u[:, :, 1:], zw], axis=2)


def kernel(depth_map, x_ray):
    B, _, H, W = depth_map.shape
    out = pl.pallas_call(
        _splat_pool_kernel,
        grid=(B,),
        in_specs=[
            pl.BlockSpec((1, 1, H, W), lambda b: (b, 0, 0, 0)),
            pl.BlockSpec((1, 1, H, W), lambda b: (b, 0, 0, 0)),
        ],
        out_specs=pl.BlockSpec((1, _D, H, W), lambda b: (b, 0, 0, 0)),
        out_shape=jax.ShapeDtypeStruct((B, _D, H, W), jnp.float32),
    )(depth_map, x_ray)
    return out[:, None]
